# Initial kernel scaffold; baseline (speedup 1.0000x reference)
import jax
import jax.numpy as jnp
from jax.experimental import pallas as pl


def kernel(rna_norm, rna_counts, rna_libsize, cas_norm, cas_counts, cas_libsize, adt_norm, adt_counts, adt_libsize, edge_index, W1, b1, W2, b2, Wg, a_src, a_dst, D1, db1, D2, db2, Wpi, bpi, Wdisp, bdisp, Wmean, bmean, Wrec, brec):
    raise NotImplementedError("write your pallas kernel here")


# speedup vs baseline: 15.8999x; 15.8999x over previous
import functools

import jax
import jax.numpy as jnp
from jax import lax
from jax.experimental import pallas as pl
from jax.experimental.pallas import tpu as pltpu
from jax.experimental.pallas import tpu_sc as plsc

N = 50000
E = 800000
D_IN = 128
D_H1 = 256
D_H2 = 128
D_LAT = 50
DX = 64
BM = 2000
GRID = N // BM

NCORE = 2
NSUB = 16
HALF = N // NCORE
HALF_PAD = 25088
EPT = E // NSUB
EB = 80
NCHUNK = EPT // EB


def _encoder_body(x_ref, w1_ref, b1_ref, w2_ref, b2_ref, waug_ref, baug_ref,
                  wdst_ref, wsrc_ref, hgx_ref, sdst_ref, smax_ref):
    x = x_ref[...]
    h = jnp.maximum(jnp.dot(x, w1_ref[...], preferred_element_type=jnp.float32)
                    + b1_ref[...], 0.0)
    h = jnp.maximum(jnp.dot(h, w2_ref[...], preferred_element_type=jnp.float32)
                    + b2_ref[...], 0.0)
    hgx_ref[...] = (jnp.dot(h, waug_ref[...], preferred_element_type=jnp.float32)
                    + baug_ref[...])
    sd = jnp.dot(h, wdst_ref[...], preferred_element_type=jnp.float32)
    sdst_ref[...] = jnp.broadcast_to(sd, (sd.shape[0], 16))
    ssrc = jnp.dot(h, wsrc_ref[...], preferred_element_type=jnp.float32)
    bmax = jnp.full((1, 1), jnp.max(ssrc), jnp.float32)

    @pl.when(pl.program_id(0) == 0)
    def _():
        smax_ref[...] = jnp.full((1, 1), -jnp.inf, jnp.float32)

    smax_ref[...] = jnp.maximum(smax_ref[...], bmax)


def _sc_edge_body(hgx_hbm, sdst_hbm, smax_hbm, src_hbm, dst_hbm, zer_hbm,
                  out_hbm,
                  smax_v, src_idx, dst_idx, rows, sdrows, lidx,
                  accum_sp, sem, sem2):
    cid = lax.axis_index("c")
    tid = lax.axis_index("s")
    lo = cid * HALF

    pltpu.sync_copy(smax_hbm, smax_v)
    smax = smax_v[...]

    pltpu.sync_copy(zer_hbm, accum_sp.at[pl.ds(tid * (HALF_PAD // NSUB),
                                               HALF_PAD // NSUB)])
    plsc.subcore_barrier()

    iota16 = lax.iota(jnp.int32, 16)
    col51 = jnp.full((16,), D_LAT + 1, jnp.int32)

    def _chunk(ci, carry):
        off = tid * EPT + ci * EB
        pltpu.sync_copy(src_hbm.at[pl.ds(off, EB)], src_idx)
        pltpu.sync_copy(dst_hbm.at[pl.ds(off, EB)], dst_idx)
        cp1 = pltpu.async_copy(hgx_hbm.at[src_idx], rows, sem)
        cp2 = pltpu.async_copy(sdst_hbm.at[dst_idx], sdrows, sem2)
        cp1.wait()
        cp2.wait()
        for j in range(EB // 16):
            dvec = dst_idx[pl.ds(j * 16, 16)]
            b = plsc.load_gather(sdrows, [iota16 + j * 16, iota16])
            a = plsc.load_gather(rows, [iota16 + j * 16, col51])
            u = a + b
            e = jnp.maximum(u, 0.2 * u)
            w = smax + b
            c = jnp.maximum(w, 0.2 * w)
            ex = jnp.exp(e - c)
            m = (dvec >= lo) & (dvec < lo + HALF)
            ex = jnp.where(m, ex, 0.0)
            li = jnp.minimum(jnp.maximum(dvec - lo, 0), HALF_PAD - 1)
            lidx[pl.ds(j * 16, 16)] = li
            for i in range(16):
                es = ex[i]
                r = j * 16 + i
                for k in range(DX // 16):
                    rows[r, pl.ds(k * 16, 16)] = rows[r, pl.ds(k * 16, 16)] * es
        pltpu.sync_copy(rows, accum_sp.at[lidx], add=True)
        return carry

    lax.fori_loop(0, NCHUNK, _chunk, 0)
    plsc.subcore_barrier()

    n_a, n_b = 1568, 1480

    @pl.when(tid < 15)
    def _():
        st = tid * n_a
        pltpu.sync_copy(accum_sp.at[pl.ds(st, n_a)],
                        out_hbm.at[pl.ds(lo + st, n_a)])

    @pl.when(tid == 15)
    def _():
        st = 15 * n_a
        pltpu.sync_copy(accum_sp.at[pl.ds(st, n_b)],
                        out_hbm.at[pl.ds(lo + st, n_b)])


def _lgamma(x):
    c = (0.99999999999980993, 676.5203681218851, -1259.1392167224028,
         771.32342877765313, -176.61502916214059, 12.507343278686905,
         -0.13857109526572012, 9.9843695780195716e-6, 1.5056327351493116e-7)
    a = jnp.full_like(x, c[0])
    for i in range(1, 9):
        a = a + c[i] / (x + (i - 1.0))
    t = x + 6.5
    return 0.9189385332046727 + jnp.log(a) + (x - 0.5) * jnp.log(t) - t


def _decoder_body(acc_ref, xn_ref, xc_ref, lib_ref,
                  d1_ref, db1_ref, d2_ref, db2_ref,
                  wpi_ref, bpi_ref, wdi_ref, bdi_ref,
                  wme_ref, bme_ref, wre_ref, bre_ref,
                  nll_ref, mse_ref):
    eps = 1e-10
    acc = acc_ref[...]
    denom = acc[:, D_LAT:D_LAT + 1]
    xl = acc * (1.0 / (denom + 1e-16))
    xl = jnp.where(xl > 0, xl, jnp.exp(jnp.minimum(xl, 0.0)) - 1.0)
    hd = jnp.maximum(jnp.dot(xl, d1_ref[...], preferred_element_type=jnp.float32)
                     + db1_ref[...], 0.0)
    hd = jnp.maximum(jnp.dot(hd, d2_ref[...], preferred_element_type=jnp.float32)
                     + db2_ref[...], 0.0)

    def head(wref, bref):
        return jnp.dot(hd, wref[...], preferred_element_type=jnp.float32) + bref[...]

    def softplus(v):
        return jnp.maximum(v, 0.0) + jnp.log1p(jnp.exp(-jnp.abs(v)))

    pi = jax.nn.sigmoid(head(wpi_ref, bpi_ref))
    disp = jnp.clip(softplus(head(wdi_ref, bdi_ref)), 1e-4, 1e4)
    mean_ = jnp.clip(softplus(head(wme_ref, bme_ref)), 1e-5, 1e6)
    recons = head(wre_ref, bre_ref)

    xn = xn_ref[...]
    xc = xc_ref[...]
    sm = mean_ * lib_ref[...]
    t1 = _lgamma(disp + eps) + _lgamma(xc + 1.0) - _lgamma(xc + disp + eps)
    t2 = ((disp + xc) * jnp.log1p(sm / (disp + eps))
          + xc * (jnp.log(disp + eps) - jnp.log(sm + eps)))
    nb_case = t1 + t2 - jnp.log(1.0 - pi + eps)
    zero_nb = jnp.exp(disp * jnp.log(disp / (disp + sm + eps)))
    zero_case = -jnp.log(pi + (1.0 - pi) * zero_nb + eps)
    res = jnp.where(xc < 1e-8, zero_case, nb_case)
    nll_blk = jnp.sum(res + 0.5 * jnp.square(pi))
    mse_blk = jnp.sum(jnp.square(recons - xn))

    @pl.when(pl.program_id(0) == 0)
    def _():
        nll_ref[...] = jnp.zeros((1, 1), jnp.float32)
        mse_ref[...] = jnp.zeros((1, 1), jnp.float32)

    nll_ref[...] += jnp.full((1, 1), nll_blk, jnp.float32)
    mse_ref[...] += jnp.full((1, 1), mse_blk, jnp.float32)


def kernel(rna_norm, rna_counts, rna_libsize, cas_norm, cas_counts, cas_libsize,
           adt_norm, adt_counts, adt_libsize, edge_index,
           W1, b1, W2, b2, Wg, a_src, a_dst,
           D1, db1, D2, db2, Wpi, bpi, Wdisp, bdisp, Wmean, bmean, Wrec, brec):
    f32 = jnp.float32
    wsrc = (Wg @ a_src).reshape(D_H2, 1)
    wdst = (Wg @ a_dst).reshape(D_H2, 1)
    waug = jnp.concatenate(
        [Wg, jnp.zeros((D_H2, 1), f32), wsrc, jnp.zeros((D_H2, DX - D_LAT - 2), f32)],
        axis=1)
    baug = jnp.zeros((1, DX), f32).at[0, D_LAT].set(1.0)

    hgx, sdst, smax = pl.pallas_call(
        _encoder_body,
        grid=(GRID,),
        in_specs=[
            pl.BlockSpec((BM, D_IN), lambda i: (i, 0)),
            pl.BlockSpec((D_IN, D_H1), lambda i: (0, 0)),
            pl.BlockSpec((1, D_H1), lambda i: (0, 0)),
            pl.BlockSpec((D_H1, D_H2), lambda i: (0, 0)),
            pl.BlockSpec((1, D_H2), lambda i: (0, 0)),
            pl.BlockSpec((D_H2, DX), lambda i: (0, 0)),
            pl.BlockSpec((1, DX), lambda i: (0, 0)),
            pl.BlockSpec((D_H2, 1), lambda i: (0, 0)),
            pl.BlockSpec((D_H2, 1), lambda i: (0, 0)),
        ],
        out_specs=[
            pl.BlockSpec((BM, DX), lambda i: (i, 0)),
            pl.BlockSpec((BM, 16), lambda i: (i, 0)),
            pl.BlockSpec((1, 1), lambda i: (0, 0)),
        ],
        out_shape=[
            jax.ShapeDtypeStruct((N, DX), f32),
            jax.ShapeDtypeStruct((N, 16), f32),
            jax.ShapeDtypeStruct((1, 1), f32),
        ],
    )(rna_norm, W1, b1.reshape(1, -1), W2, b2.reshape(1, -1), waug, baug,
      wdst, wsrc)

    smax16 = jnp.broadcast_to(smax.reshape(1), (16,))
    zer = jnp.zeros((HALF_PAD // NSUB, DX), f32)

    mesh = plsc.VectorSubcoreMesh(core_axis_name="c", subcore_axis_name="s")
    accum = pl.kernel(
        _sc_edge_body,
        out_type=jax.ShapeDtypeStruct((N, DX), f32),
        mesh=mesh,
        compiler_params=pltpu.CompilerParams(needs_layout_passes=False,
                                             use_tc_tiling_on_sc=False),
        scratch_types=[
            pltpu.VMEM((16,), f32),
            pltpu.VMEM((EB,), jnp.int32),
            pltpu.VMEM((EB,), jnp.int32),
            pltpu.VMEM((EB, DX), f32),
            pltpu.VMEM((EB, 16), f32),
            pltpu.VMEM((EB,), jnp.int32),
            pltpu.VMEM_SHARED((HALF_PAD, DX), f32),
            pltpu.SemaphoreType.DMA,
            pltpu.SemaphoreType.DMA,
        ],
    )(hgx, sdst, smax16, edge_index[0], edge_index[1], zer)

    d1p = jnp.concatenate([D1, jnp.zeros((DX - D_LAT, D_H2), f32)], axis=0)
    nll_sum, mse_sum = pl.pallas_call(
        _decoder_body,
        grid=(GRID,),
        in_specs=[
            pl.BlockSpec((BM, DX), lambda i: (i, 0)),
            pl.BlockSpec((BM, D_IN), lambda i: (i, 0)),
            pl.BlockSpec((BM, D_IN), lambda i: (i, 0)),
            pl.BlockSpec((BM, 1), lambda i: (i, 0)),
            pl.BlockSpec((DX, D_H2), lambda i: (0, 0)),
            pl.BlockSpec((1, D_H2), lambda i: (0, 0)),
            pl.BlockSpec((D_H2, D_H1), lambda i: (0, 0)),
            pl.BlockSpec((1, D_H1), lambda i: (0, 0)),
            pl.BlockSpec((D_H1, D_IN), lambda i: (0, 0)),
            pl.BlockSpec((1, D_IN), lambda i: (0, 0)),
            pl.BlockSpec((D_H1, D_IN), lambda i: (0, 0)),
            pl.BlockSpec((1, D_IN), lambda i: (0, 0)),
            pl.BlockSpec((D_H1, D_IN), lambda i: (0, 0)),
            pl.BlockSpec((1, D_IN), lambda i: (0, 0)),
            pl.BlockSpec((D_H1, D_IN), lambda i: (0, 0)),
            pl.BlockSpec((1, D_IN), lambda i: (0, 0)),
        ],
        out_specs=[
            pl.BlockSpec((1, 1), lambda i: (0, 0)),
            pl.BlockSpec((1, 1), lambda i: (0, 0)),
        ],
        out_shape=[
            jax.ShapeDtypeStruct((1, 1), f32),
            jax.ShapeDtypeStruct((1, 1), f32),
        ],
    )(accum, rna_norm, rna_counts, rna_libsize,
      d1p, db1.reshape(1, -1), D2, db2.reshape(1, -1),
      Wpi, bpi.reshape(1, -1), Wdisp, bdisp.reshape(1, -1),
      Wmean, bmean.reshape(1, -1), Wrec, brec.reshape(1, -1))

    scale = 1.0 / (N * D_IN)
    return (nll_sum[0, 0] * scale, mse_sum[0, 0] * scale)


# speedup vs baseline: 30.2756x; 1.9041x over previous
import functools

import jax
import jax.numpy as jnp
from jax import lax
from jax.experimental import pallas as pl
from jax.experimental.pallas import tpu as pltpu
from jax.experimental.pallas import tpu_sc as plsc

N = 50000
E = 800000
D_IN = 128
D_H1 = 256
D_H2 = 128
D_LAT = 50
DX = 64
BM = 2000
GRID = N // BM

NCORE = 2
NSUB = 16
HALF = N // NCORE
HALF_PAD = 25088
EPT = E // NSUB
EB = 80
NCHUNK = EPT // EB


def _encoder_body(x_ref, w1_ref, b1_ref, w2_ref, b2_ref, waug_ref, baug_ref,
                  wdst_ref, wsrc_ref, hgx_ref, sdst_ref, smax_ref):
    x = x_ref[...]
    h = jnp.maximum(jnp.dot(x, w1_ref[...], preferred_element_type=jnp.float32)
                    + b1_ref[...], 0.0)
    h = jnp.maximum(jnp.dot(h, w2_ref[...], preferred_element_type=jnp.float32)
                    + b2_ref[...], 0.0)
    hgx_ref[...] = (jnp.dot(h, waug_ref[...], preferred_element_type=jnp.float32)
                    + baug_ref[...])
    sd = jnp.dot(h, wdst_ref[...], preferred_element_type=jnp.float32)
    sdst_ref[...] = jnp.broadcast_to(sd, (sd.shape[0], 16))
    ssrc = jnp.dot(h, wsrc_ref[...], preferred_element_type=jnp.float32)
    bmax = jnp.full((1, 1), jnp.max(ssrc), jnp.float32)

    @pl.when(pl.program_id(0) == 0)
    def _():
        smax_ref[...] = jnp.full((1, 1), -jnp.inf, jnp.float32)

    smax_ref[...] = jnp.maximum(smax_ref[...], bmax)


def _sc_edge_body(hgx_hbm, sdst_hbm, smax_hbm, src_hbm, dst_hbm, zer_hbm,
                  out_hbm,
                  smax_v, src_i, dst_i, row_b, sd_b, li_b,
                  accum_sp, sem_si, sem_di, sem_r, sem_s, sem_w):
    cid = lax.axis_index("c")
    tid = lax.axis_index("s")
    lo = cid * HALF

    pltpu.sync_copy(smax_hbm, smax_v)
    smax = smax_v[...]

    pltpu.sync_copy(zer_hbm, accum_sp.at[pl.ds(tid * (HALF_PAD // NSUB),
                                               HALF_PAD // NSUB)])
    plsc.subcore_barrier()

    iota16 = lax.iota(jnp.int32, 16)
    col51 = jnp.full((16,), D_LAT + 1, jnp.int32)
    base = tid * EPT

    def fire_idx(c, b):
        off = base + c * EB
        pltpu.async_copy(src_hbm.at[pl.ds(off, EB)], src_i.at[b], sem_si.at[b])
        pltpu.async_copy(dst_hbm.at[pl.ds(off, EB)], dst_i.at[b], sem_di.at[b])

    def wait_idx(b):
        pltpu.make_async_copy(src_hbm.at[pl.ds(0, EB)], src_i.at[b],
                              sem_si.at[b]).wait()
        pltpu.make_async_copy(dst_hbm.at[pl.ds(0, EB)], dst_i.at[b],
                              sem_di.at[b]).wait()

    def fire_gather(b):
        pltpu.async_copy(hgx_hbm.at[src_i.at[b]], row_b.at[b], sem_r.at[b])
        pltpu.async_copy(sdst_hbm.at[dst_i.at[b]], sd_b.at[b], sem_s.at[b])

    def wait_gather(b):
        pltpu.make_async_copy(hgx_hbm.at[src_i.at[b]], row_b.at[b],
                              sem_r.at[b]).wait()
        pltpu.make_async_copy(sdst_hbm.at[dst_i.at[b]], sd_b.at[b],
                              sem_s.at[b]).wait()

    def fire_scatter(b):
        pltpu.async_copy(row_b.at[b], accum_sp.at[li_b.at[b]], sem_w.at[b],
                         add=True)

    def wait_scatter(b):
        pltpu.make_async_copy(row_b.at[b], accum_sp.at[li_b.at[b]],
                              sem_w.at[b]).wait()

    def compute(b):
        rows = row_b.at[b]
        for j in range(EB // 16):
            dvec = dst_i[b, pl.ds(j * 16, 16)]
            bb = plsc.load_gather(sd_b.at[b], [iota16 + j * 16, iota16])
            a = plsc.load_gather(rows, [iota16 + j * 16, col51])
            u = a + bb
            e = jnp.maximum(u, 0.2 * u)
            w = smax + bb
            c = jnp.maximum(w, 0.2 * w)
            ex = jnp.exp(e - c)
            m = (dvec >= lo) & (dvec < lo + HALF)
            ex = jnp.where(m, ex, 0.0)
            li = jnp.minimum(jnp.maximum(dvec - lo, 0), HALF_PAD - 1)
            li_b[b, pl.ds(j * 16, 16)] = li
            for i in range(16):
                es = ex[i]
                r = j * 16 + i
                for k in range(DX // 16):
                    rows[r, pl.ds(k * 16, 16)] = rows[r, pl.ds(k * 16, 16)] * es

    fire_idx(0, 0)
    fire_idx(1, 1)
    wait_idx(0)
    fire_gather(0)

    def _pair(i, carry):
        for b in range(2):
            c = 2 * i + b
            o = 1 - b

            @pl.when(c > 0)
            def _():
                wait_scatter(o)

            @pl.when(c < NCHUNK - 1)
            def _():
                wait_idx(o)
                fire_gather(o)
            wait_gather(b)
            compute(b)

            @pl.when(c < NCHUNK - 2)
            def _():
                fire_idx(c + 2, b)
            fire_scatter(b)
        return carry

    lax.fori_loop(0, NCHUNK // 2, _pair, 0)
    wait_scatter(1)
    wait_gather(0)
    compute(0)
    fire_scatter(0)
    wait_scatter(0)
    plsc.subcore_barrier()

    n_a, n_b = 1568, 1480

    @pl.when(tid < 15)
    def _():
        st = tid * n_a
        pltpu.sync_copy(accum_sp.at[pl.ds(st, n_a)],
                        out_hbm.at[pl.ds(lo + st, n_a)])

    @pl.when(tid == 15)
    def _():
        st = 15 * n_a
        pltpu.sync_copy(accum_sp.at[pl.ds(st, n_b)],
                        out_hbm.at[pl.ds(lo + st, n_b)])


def _lgamma(x):
    c = (0.99999999999980993, 676.5203681218851, -1259.1392167224028,
         771.32342877765313, -176.61502916214059, 12.507343278686905,
         -0.13857109526572012, 9.9843695780195716e-6, 1.5056327351493116e-7)
    a = jnp.full_like(x, c[0])
    for i in range(1, 9):
        a = a + c[i] / (x + (i - 1.0))
    t = x + 6.5
    return 0.9189385332046727 + jnp.log(a) + (x - 0.5) * jnp.log(t) - t


def _decoder_body(acc_ref, xn_ref, xc_ref, lib_ref,
                  d1_ref, db1_ref, d2_ref, db2_ref,
                  wpi_ref, bpi_ref, wdi_ref, bdi_ref,
                  wme_ref, bme_ref, wre_ref, bre_ref,
                  nll_ref, mse_ref):
    eps = 1e-10
    acc = acc_ref[...]
    denom = acc[:, D_LAT:D_LAT + 1]
    xl = acc * (1.0 / (denom + 1e-16))
    xl = jnp.where(xl > 0, xl, jnp.exp(jnp.minimum(xl, 0.0)) - 1.0)
    hd = jnp.maximum(jnp.dot(xl, d1_ref[...], preferred_element_type=jnp.float32)
                     + db1_ref[...], 0.0)
    hd = jnp.maximum(jnp.dot(hd, d2_ref[...], preferred_element_type=jnp.float32)
                     + db2_ref[...], 0.0)

    def head(wref, bref):
        return jnp.dot(hd, wref[...], preferred_element_type=jnp.float32) + bref[...]

    def softplus(v):
        return jnp.maximum(v, 0.0) + jnp.log1p(jnp.exp(-jnp.abs(v)))

    pi = jax.nn.sigmoid(head(wpi_ref, bpi_ref))
    disp = jnp.clip(softplus(head(wdi_ref, bdi_ref)), 1e-4, 1e4)
    mean_ = jnp.clip(softplus(head(wme_ref, bme_ref)), 1e-5, 1e6)
    recons = head(wre_ref, bre_ref)

    xn = xn_ref[...]
    xc = xc_ref[...]
    sm = mean_ * lib_ref[...]
    t1 = _lgamma(disp + eps) + _lgamma(xc + 1.0) - _lgamma(xc + disp + eps)
    t2 = ((disp + xc) * jnp.log1p(sm / (disp + eps))
          + xc * (jnp.log(disp + eps) - jnp.log(sm + eps)))
    nb_case = t1 + t2 - jnp.log(1.0 - pi + eps)
    zero_nb = jnp.exp(disp * jnp.log(disp / (disp + sm + eps)))
    zero_case = -jnp.log(pi + (1.0 - pi) * zero_nb + eps)
    res = jnp.where(xc < 1e-8, zero_case, nb_case)
    nll_blk = jnp.sum(res + 0.5 * jnp.square(pi))
    mse_blk = jnp.sum(jnp.square(recons - xn))

    @pl.when(pl.program_id(0) == 0)
    def _():
        nll_ref[...] = jnp.zeros((1, 1), jnp.float32)
        mse_ref[...] = jnp.zeros((1, 1), jnp.float32)

    nll_ref[...] += jnp.full((1, 1), nll_blk, jnp.float32)
    mse_ref[...] += jnp.full((1, 1), mse_blk, jnp.float32)


def kernel(rna_norm, rna_counts, rna_libsize, cas_norm, cas_counts, cas_libsize,
           adt_norm, adt_counts, adt_libsize, edge_index,
           W1, b1, W2, b2, Wg, a_src, a_dst,
           D1, db1, D2, db2, Wpi, bpi, Wdisp, bdisp, Wmean, bmean, Wrec, brec):
    f32 = jnp.float32
    wsrc = (Wg @ a_src).reshape(D_H2, 1)
    wdst = (Wg @ a_dst).reshape(D_H2, 1)
    waug = jnp.concatenate(
        [Wg, jnp.zeros((D_H2, 1), f32), wsrc, jnp.zeros((D_H2, DX - D_LAT - 2), f32)],
        axis=1)
    baug = jnp.zeros((1, DX), f32).at[0, D_LAT].set(1.0)

    hgx, sdst, smax = pl.pallas_call(
        _encoder_body,
        grid=(GRID,),
        in_specs=[
            pl.BlockSpec((BM, D_IN), lambda i: (i, 0)),
            pl.BlockSpec((D_IN, D_H1), lambda i: (0, 0)),
            pl.BlockSpec((1, D_H1), lambda i: (0, 0)),
            pl.BlockSpec((D_H1, D_H2), lambda i: (0, 0)),
            pl.BlockSpec((1, D_H2), lambda i: (0, 0)),
            pl.BlockSpec((D_H2, DX), lambda i: (0, 0)),
            pl.BlockSpec((1, DX), lambda i: (0, 0)),
            pl.BlockSpec((D_H2, 1), lambda i: (0, 0)),
            pl.BlockSpec((D_H2, 1), lambda i: (0, 0)),
        ],
        out_specs=[
            pl.BlockSpec((BM, DX), lambda i: (i, 0)),
            pl.BlockSpec((BM, 16), lambda i: (i, 0)),
            pl.BlockSpec((1, 1), lambda i: (0, 0)),
        ],
        out_shape=[
            jax.ShapeDtypeStruct((N, DX), f32),
            jax.ShapeDtypeStruct((N, 16), f32),
            jax.ShapeDtypeStruct((1, 1), f32),
        ],
    )(rna_norm, W1, b1.reshape(1, -1), W2, b2.reshape(1, -1), waug, baug,
      wdst, wsrc)

    smax16 = jnp.broadcast_to(smax.reshape(1), (16,))
    zer = jnp.zeros((HALF_PAD // NSUB, DX), f32)

    mesh = plsc.VectorSubcoreMesh(core_axis_name="c", subcore_axis_name="s")
    accum = pl.kernel(
        _sc_edge_body,
        out_type=jax.ShapeDtypeStruct((N, DX), f32),
        mesh=mesh,
        compiler_params=pltpu.CompilerParams(needs_layout_passes=False,
                                             use_tc_tiling_on_sc=False),
        scratch_types=[
            pltpu.VMEM((16,), f32),
            pltpu.VMEM((2, EB), jnp.int32),
            pltpu.VMEM((2, EB), jnp.int32),
            pltpu.VMEM((2, EB, DX), f32),
            pltpu.VMEM((2, EB, 16), f32),
            pltpu.VMEM((2, EB), jnp.int32),
            pltpu.VMEM_SHARED((HALF_PAD, DX), f32),
            pltpu.SemaphoreType.DMA((2,)),
            pltpu.SemaphoreType.DMA((2,)),
            pltpu.SemaphoreType.DMA((2,)),
            pltpu.SemaphoreType.DMA((2,)),
            pltpu.SemaphoreType.DMA((2,)),
        ],
    )(hgx, sdst, smax16, edge_index[0], edge_index[1], zer)

    d1p = jnp.concatenate([D1, jnp.zeros((DX - D_LAT, D_H2), f32)], axis=0)
    nll_sum, mse_sum = pl.pallas_call(
        _decoder_body,
        grid=(GRID,),
        in_specs=[
            pl.BlockSpec((BM, DX), lambda i: (i, 0)),
            pl.BlockSpec((BM, D_IN), lambda i: (i, 0)),
            pl.BlockSpec((BM, D_IN), lambda i: (i, 0)),
            pl.BlockSpec((BM, 1), lambda i: (i, 0)),
            pl.BlockSpec((DX, D_H2), lambda i: (0, 0)),
            pl.BlockSpec((1, D_H2), lambda i: (0, 0)),
            pl.BlockSpec((D_H2, D_H1), lambda i: (0, 0)),
            pl.BlockSpec((1, D_H1), lambda i: (0, 0)),
            pl.BlockSpec((D_H1, D_IN), lambda i: (0, 0)),
            pl.BlockSpec((1, D_IN), lambda i: (0, 0)),
            pl.BlockSpec((D_H1, D_IN), lambda i: (0, 0)),
            pl.BlockSpec((1, D_IN), lambda i: (0, 0)),
            pl.BlockSpec((D_H1, D_IN), lambda i: (0, 0)),
            pl.BlockSpec((1, D_IN), lambda i: (0, 0)),
            pl.BlockSpec((D_H1, D_IN), lambda i: (0, 0)),
            pl.BlockSpec((1, D_IN), lambda i: (0, 0)),
        ],
        out_specs=[
            pl.BlockSpec((1, 1), lambda i: (0, 0)),
            pl.BlockSpec((1, 1), lambda i: (0, 0)),
        ],
        out_shape=[
            jax.ShapeDtypeStruct((1, 1), f32),
            jax.ShapeDtypeStruct((1, 1), f32),
        ],
    )(accum, rna_norm, rna_counts, rna_libsize,
      d1p, db1.reshape(1, -1), D2, db2.reshape(1, -1),
      Wpi, bpi.reshape(1, -1), Wdisp, bdisp.reshape(1, -1),
      Wmean, bmean.reshape(1, -1), Wrec, brec.reshape(1, -1))

    scale = 1.0 / (N * D_IN)
    return (nll_sum[0, 0] * scale, mse_sum[0, 0] * scale)
